# baseline (speedup 1.0000x reference)
import jax
import jax.numpy as jnp
from jax import lax
from jax.experimental import pallas as pl

_B = 128
_n = 32
_F = 128
_H = 64
_O = 128
_G = 16


def _body(feat_ref, x_ref, w1_ref, b1_ref, w2_ref, b2_ref, w3_ref, b3_ref,
          sel_ref, out_ref):
    bf = jnp.bfloat16
    f32 = jnp.float32
    h = feat_ref[...].astype(bf)
    x = x_ref[...]
    w1 = w1_ref[...]
    w1dh = 0.5 * w1[0:1, :]
    w1dm = (-w1[0:1, :]).astype(bf)
    w1a = (0.5 * w1[1:1 + _F, :]).astype(bf)
    w1b = (0.5 * w1[1 + _F:, :]).astype(bf)
    a = jnp.dot(h, w1a, preferred_element_type=f32)
    c = jnp.dot(h, w1b, preferred_element_type=f32)
    xx = x * x
    sqn = xx[:, 0:1] + xx[:, 1:2] + xx[:, 2:3]
    ap = (a + sqn * w1dh).astype(bf)
    cp = (c + sqn * w1dh + 0.5 * b1_ref[...]).astype(bf)
    x3 = x.astype(bf).reshape(_G, _n, 3)
    d2c = lax.dot_general(x3, x3, (((2,), (2,)), ((0,), (0,))),
                          preferred_element_type=f32)
    d2cb = d2c.astype(bf)
    zh = (ap.reshape(_G, 1, _n, _H)
          + cp.reshape(_G, _n, 1, _H)
          + d2cb[:, :, :, None] * w1dm[0])
    t1 = jnp.tanh(zh)
    h1 = (zh + zh * t1).reshape(_G * _n * _n, _H)
    w2h = (0.5 * w2_ref[...]).astype(bf)
    z2h = (jnp.dot(h1, w2h, preferred_element_type=f32)).astype(bf)
    z2h = z2h + (0.5 * b2_ref[...][0]).astype(bf)
    t2 = jnp.tanh(z2h)
    h2 = z2h + z2h * t2
    s = jnp.dot(sel_ref[...], h2,
                preferred_element_type=f32)
    out_ref[...] = (jnp.dot(s, w3_ref[...], preferred_element_type=f32)
                    + float(_n * _n) * b3_ref[...][0])


def kernel(feat, coordinate, w1, b1, w2, b2, w3, b3, num_graphs):
    del num_graphs
    sel = jnp.repeat(jnp.eye(_G, dtype=jnp.bfloat16), _n * _n, axis=1)
    b1r = b1.reshape(1, _H)
    b2r = b2.reshape(1, _H)
    b3r = b3.reshape(1, _O)

    grid = _B // _G
    out = pl.pallas_call(
        _body,
        grid=(grid,),
        in_specs=[
            pl.BlockSpec((_G * _n, _F), lambda g: (g, 0)),
            pl.BlockSpec((_G * _n, 3), lambda g: (g, 0)),
            pl.BlockSpec((1 + 2 * _F, _H), lambda g: (0, 0)),
            pl.BlockSpec((1, _H), lambda g: (0, 0)),
            pl.BlockSpec((_H, _H), lambda g: (0, 0)),
            pl.BlockSpec((1, _H), lambda g: (0, 0)),
            pl.BlockSpec((_H, _O), lambda g: (0, 0)),
            pl.BlockSpec((1, _O), lambda g: (0, 0)),
            pl.BlockSpec((_G, _G * _n * _n), lambda g: (0, 0)),
        ],
        out_specs=pl.BlockSpec((_G, _O), lambda g: (g, 0)),
        out_shape=jax.ShapeDtypeStruct((_B, _O), jnp.float32),
    )(feat, coordinate, w1, b1r, w2, b2r, w3, b3r, sel)
    return out


# speedup vs baseline: 1.0229x; 1.0229x over previous
import jax
import jax.numpy as jnp
from jax import lax
from jax.experimental import pallas as pl

_B = 128
_n = 32
_F = 128
_H = 64
_O = 128
_G = 32


def _body(feat_ref, x_ref, w1_ref, b1_ref, w2_ref, b2_ref, w3_ref, b3_ref,
          sel_ref, out_ref):
    bf = jnp.bfloat16
    f32 = jnp.float32
    h = feat_ref[...].astype(bf)
    x = x_ref[...]
    w1 = w1_ref[...]
    w1dh = 0.5 * w1[0:1, :]
    w1dm = (-w1[0:1, :]).astype(bf)
    w1a = (0.5 * w1[1:1 + _F, :]).astype(bf)
    w1b = (0.5 * w1[1 + _F:, :]).astype(bf)
    a = jnp.dot(h, w1a, preferred_element_type=f32)
    c = jnp.dot(h, w1b, preferred_element_type=f32)
    xx = x * x
    sqn = xx[:, 0:1] + xx[:, 1:2] + xx[:, 2:3]
    ap = (a + sqn * w1dh).astype(bf)
    cp = (c + sqn * w1dh + 0.5 * b1_ref[...]).astype(bf)
    x3 = x.astype(bf).reshape(_G, _n, 3)
    d2c = lax.dot_general(x3, x3, (((2,), (2,)), ((0,), (0,))),
                          preferred_element_type=f32)
    d2cb = d2c.astype(bf)
    zh = (ap.reshape(_G, 1, _n, _H)
          + cp.reshape(_G, _n, 1, _H)
          + d2cb[:, :, :, None] * w1dm[0])
    t1 = jnp.tanh(zh)
    h1 = (zh + zh * t1).reshape(_G * _n * _n, _H)
    w2h = (0.5 * w2_ref[...]).astype(bf)
    z2h = (jnp.dot(h1, w2h, preferred_element_type=f32)).astype(bf)
    z2h = z2h + (0.5 * b2_ref[...][0]).astype(bf)
    t2 = jnp.tanh(z2h)
    h2 = z2h + z2h * t2
    s = jnp.dot(sel_ref[...], h2,
                preferred_element_type=f32)
    out_ref[...] = (jnp.dot(s, w3_ref[...], preferred_element_type=f32)
                    + float(_n * _n) * b3_ref[...][0])


def kernel(feat, coordinate, w1, b1, w2, b2, w3, b3, num_graphs):
    del num_graphs
    sel = jnp.repeat(jnp.eye(_G, dtype=jnp.bfloat16), _n * _n, axis=1)
    b1r = b1.reshape(1, _H)
    b2r = b2.reshape(1, _H)
    b3r = b3.reshape(1, _O)

    grid = _B // _G
    out = pl.pallas_call(
        _body,
        grid=(grid,),
        in_specs=[
            pl.BlockSpec((_G * _n, _F), lambda g: (g, 0)),
            pl.BlockSpec((_G * _n, 3), lambda g: (g, 0)),
            pl.BlockSpec((1 + 2 * _F, _H), lambda g: (0, 0)),
            pl.BlockSpec((1, _H), lambda g: (0, 0)),
            pl.BlockSpec((_H, _H), lambda g: (0, 0)),
            pl.BlockSpec((1, _H), lambda g: (0, 0)),
            pl.BlockSpec((_H, _O), lambda g: (0, 0)),
            pl.BlockSpec((1, _O), lambda g: (0, 0)),
            pl.BlockSpec((_G, _G * _n * _n), lambda g: (0, 0)),
        ],
        out_specs=pl.BlockSpec((_G, _O), lambda g: (g, 0)),
        out_shape=jax.ShapeDtypeStruct((_B, _O), jnp.float32),
    )(feat, coordinate, w1, b1r, w2, b2r, w3, b3r, sel)
    return out
